# baseline (speedup 1.0000x reference)
import jax
import jax.numpy as jnp
from jax import lax
from jax.experimental import pallas as pl
from jax.experimental.pallas import tpu as pltpu
from jax.experimental.pallas import tpu_sc as plsc

_B, _N, _D, _K = 32, 32768, 16, 128
_L = 16
_NV = _N // _L
_NBINS = 256
_NC, _NS = 2, 16
_MININT = -(2 ** 31)
_FLIP = 0x7FFFFFFF


def _mono_key(v):
    bits = lax.bitcast_convert_type(v, jnp.int32)
    return jnp.where(bits >= 0, bits, bits ^ jnp.int32(_FLIP))


def _inv_key(key):
    bits = jnp.where(key >= 0, key, key ^ jnp.int32(_FLIP))
    return lax.bitcast_convert_type(bits, jnp.float32)


def _digit(key, shift):
    ub = key ^ jnp.int32(_MININT)
    return lax.shift_right_logical(ub, shift) & jnp.int32(0xFF)


def _body(scores_hbm, feats_hbm, out_hbm,
          scores_v, ck, ci, hist, summ, wl,
          gidx, gidx16, valsf, rows_t, sem):
    lanes = jnp.arange(_L, dtype=jnp.int32)
    zeros16 = jnp.zeros((_L,), jnp.int32)
    ones16 = jnp.ones((_L,), jnp.int32)
    b = lax.axis_index("c") * _NS + lax.axis_index("s")

    pltpu.sync_copy(
        scores_hbm.at[lax.div(b, jnp.int32(8)), :, lax.rem(b, jnp.int32(8)), :],
        scores_v)

    def zero_hist():
        def zbody(j, c):
            hist[pl.ds(j * _L, _L)] = zeros16
            return c
        lax.fori_loop(0, _NBINS, zbody, 0, unroll=8)

    def find_dstar(k_rem):
        gsum = []
        for g in range(16):
            acc = hist[pl.ds(g * 256, _L)]
            for t in range(1, 16):
                acc = acc + hist[pl.ds(g * 256 + t * _L, _L)]
            gsum.append(jnp.sum(acc))
        cum = jnp.int32(0)
        gstar = jnp.int32(0)
        above_g = jnp.int32(0)
        found = jnp.bool_(False)
        for g in range(15, -1, -1):
            new_cum = cum + gsum[g]
            crossed = jnp.logical_and(jnp.logical_not(found), new_cum >= k_rem)
            gstar = jnp.where(crossed, g, gstar)
            above_g = jnp.where(crossed, cum, above_g)
            found = jnp.logical_or(found, crossed)
            cum = new_cum

        def fbody(jj, carry):
            cum_f, dstar, above, found_f = carry
            j = _L - 1 - jj
            tot = jnp.sum(hist[pl.ds(gstar * 256 + j * _L, _L)])
            new_cum = cum_f + tot
            crossed = jnp.logical_and(jnp.logical_not(found_f),
                                      new_cum >= k_rem)
            dstar = jnp.where(crossed, gstar * _L + j, dstar)
            above = jnp.where(crossed, cum_f, above)
            found_f = jnp.logical_or(found_f, crossed)
            return (new_cum, dstar, above, found_f)
        _, dstar, above, _ = lax.fori_loop(
            0, _L, fbody,
            (above_g, jnp.int32(0), jnp.int32(0), jnp.bool_(False)), unroll=4)
        return dstar, above

    zero_hist()

    @plsc.parallel_loop(0, 256, unroll=2)
    def _h0(r):
        for col in range(8):
            key = _mono_key(scores_v[r, pl.ds(col * _L, _L)])
            d = _digit(key, 24)
            plsc.addupdate_scatter(hist, [d * _L + lanes], ones16)
            cm = plsc.cummax(key)
            plsc.store_scatter(summ, [jnp.full((_L,), r * 8 + col, jnp.int32)],
                               cm, mask=lanes == _L - 1)

    d0star, above0 = find_dstar(jnp.int32(_K))
    k_rem = jnp.int32(_K) - above0

    @plsc.parallel_loop(0, _NV // _L, unroll=2, carry=zeros16)
    def nwl_v(w, nwl):
        mx = summ[pl.ds(w * _L, _L)]
        m = _digit(mx, 24) >= d0star
        kp = jnp.where(m, 1, 0)
        pos = nwl + plsc.cumsum(kp) - kp
        plsc.store_scatter(wl, [pos], w * _L + lanes, mask=m)
        return nwl + plsc.all_reduce_population_count(m)
    nwl = jnp.max(nwl_v)

    @plsc.parallel_loop(0, nwl, unroll=2, carry=zeros16)
    def cnt(w, cnt_c):
        vid = plsc.load_gather(wl, [jnp.full((_L,), w, jnp.int32)])
        row = lax.shift_right_logical(vid, 3)
        colb = (vid & jnp.int32(7)) * _L
        key = _mono_key(plsc.load_gather(scores_v, [row, colb + lanes]))
        m = _digit(key, 24) >= d0star
        addr = cnt_c * _L + lanes
        plsc.store_scatter(ck, [addr], key, mask=m)
        plsc.store_scatter(ci, [addr], vid * _L + lanes, mask=m)
        return cnt_c + jnp.where(m, 1, 0)
    nrows = jnp.max(cnt)

    zero_hist()

    @plsc.parallel_loop(0, nrows, unroll=2)
    def _h1(q):
        key = ck[pl.ds(q * _L, _L)]
        match = (q < cnt) & (_digit(key, 24) == d0star)
        plsc.addupdate_scatter(hist, [_digit(key, 16) * _L + lanes],
                               ones16, mask=match)

    d1star, above1 = find_dstar(k_rem)

    @plsc.parallel_loop(0, nrows, unroll=2, carry=zeros16)
    def m1_v(q, nglob):
        key = ck[pl.ds(q * _L, _L)]
        iv = ci[pl.ds(q * _L, _L)]
        d0 = _digit(key, 24)
        keep = (q < cnt) & ((d0 > d0star) |
                            ((d0 == d0star) & (_digit(key, 16) >= d1star)))
        kp = jnp.where(keep, 1, 0)
        pos = nglob + plsc.cumsum(kp) - kp
        plsc.store_scatter(ck, [pos], key, mask=keep)
        plsc.store_scatter(ci, [pos], iv, mask=keep)
        return nglob + plsc.all_reduce_population_count(keep)
    m1 = jnp.max(m1_v)
    nrows1 = lax.div(m1 + jnp.int32(_L - 1), jnp.int32(_L))

    _NR = 12

    def _rank_fast(_):
        krows = [ck[pl.ds(p * _L, _L)] for p in range(_NR)]
        irows = [ci[pl.ds(p * _L, _L)] for p in range(_NR)]

        @plsc.parallel_loop(0, m1, unroll=2,
                            carry=tuple(zeros16 for _ in range(_NR)))
        def ranks(j, rk):
            jv = jnp.full((_L,), j, jnp.int32)
            kj = plsc.load_gather(ck, [jv])
            ij = plsc.load_gather(ci, [jv])
            return tuple(
                rk[p] + jnp.where(
                    (kj > krows[p]) | ((kj == krows[p]) & (ij < irows[p])),
                    1, 0)
                for p in range(_NR))
        for p in range(_NR):
            mw = ((p * _L + lanes) < m1_v) & (ranks[p] < _K)
            plsc.store_scatter(gidx, [ranks[p]], irows[p], mask=mw)
            plsc.store_scatter(valsf, [ranks[p]], _inv_key(krows[p]), mask=mw)
        return 0

    def _rank_slow(_):
        @plsc.parallel_loop(0, m1, unroll=2)
        def _rank_one(s):
            sv = jnp.full((_L,), s, jnp.int32)
            kiv = plsc.load_gather(ck, [sv])
            iiv = plsc.load_gather(ci, [sv])

            def inner(q, rank):
                kq = ck[pl.ds(q * _L, _L)]
                iq = ci[pl.ds(q * _L, _L)]
                vq = (q * _L + lanes) < m1_v
                m = vq & ((kq > kiv) | ((kq == kiv) & (iq < iiv)))
                return rank + plsc.all_reduce_population_count(m)
            rank = lax.fori_loop(0, nrows1, inner, zeros16)

            mw = (rank < _K) & (lanes == 0)
            plsc.store_scatter(gidx, [rank], iiv, mask=mw)
            plsc.store_scatter(valsf, [rank], _inv_key(kiv), mask=mw)
        return 0

    lax.cond(m1 <= _NR * _L, _rank_fast, _rank_slow, 0)

    for blk in range(_K // _L):
        v = gidx[pl.ds(blk * _L, _L)]
        hi = lax.shift_left(lax.shift_right_logical(v, 7), 10) + \
            (v & jnp.int32(127))
        for j in range(_D):
            base = b * jnp.int32(1 << 19) + jnp.int32(
                (j // 8) * (1 << 18) + (j % 8) * 128)
            gidx16[pl.ds(j * _K + blk * _L, _L)] = hi + base
    descs = []
    for j in range(_D):
        descs.append(pltpu.async_copy(
            feats_hbm.at[gidx16.at[pl.ds(j * _K, _K)]],
            rows_t.at[pl.ds(j * _K, _K)], sem))
    for d in descs:
        d.wait()

    for j in range(_D):
        for blk in range(_K // _L):
            sl = pl.ds(j * _K + blk * _L, _L)
            rows_t[sl] = rows_t[sl] * valsf[pl.ds(blk * _L, _L)]

    pltpu.sync_copy(rows_t, out_hbm.at[pl.ds(b * _D * _K, _D * _K)])


@jax.jit
def _run(scores, feats2d):
    mesh = plsc.VectorSubcoreMesh(core_axis_name="c", subcore_axis_name="s",
                                  num_cores=_NC, num_subcores=_NS)
    fn = pl.kernel(
        _body,
        out_type=jax.ShapeDtypeStruct((_B * _D * _K,), jnp.float32),
        mesh=mesh,
        compiler_params=pltpu.CompilerParams(needs_layout_passes=False,
                                             use_tc_tiling_on_sc=False),
        scratch_types=[
            pltpu.VMEM((256, 128), jnp.float32),
            pltpu.VMEM((_N,), jnp.int32),
            pltpu.VMEM((_N,), jnp.int32),
            pltpu.VMEM((_NBINS * _L,), jnp.int32),
            pltpu.VMEM((_NV,), jnp.int32),
            pltpu.VMEM((_NV,), jnp.int32),
            pltpu.VMEM((_K,), jnp.int32),
            pltpu.VMEM((_D * _K,), jnp.int32),
            pltpu.VMEM((_K,), jnp.float32),
            pltpu.VMEM((_D * _K,), jnp.float32),
            pltpu.SemaphoreType.DMA,
        ],
    )
    return fn(scores, feats2d)


def kernel(scores, score_feats, k):
    del k
    scores_t = scores.reshape(4, 8, 256, 128).transpose(0, 2, 1, 3)
    feats_l = (score_feats.transpose(0, 2, 1)
               .reshape(_B, 2, 8, 256, 128)
               .transpose(0, 1, 3, 2, 4)
               .reshape(-1))
    flat = _run(scores_t, feats_l)
    return flat.reshape(_B, _D, _K).transpose(0, 2, 1)


# speedup vs baseline: 1.0625x; 1.0625x over previous
import jax
import jax.numpy as jnp
from jax import lax
from jax.experimental import pallas as pl
from jax.experimental.pallas import tpu as pltpu
from jax.experimental.pallas import tpu_sc as plsc

_B, _N, _D, _K = 32, 32768, 16, 128
_L = 16
_NV = _N // _L
_NBINS = 256
_NC, _NS = 2, 16
_MININT = -(2 ** 31)
_FLIP = 0x7FFFFFFF


def _mono_key(v):
    bits = lax.bitcast_convert_type(v, jnp.int32)
    return jnp.where(bits >= 0, bits, bits ^ jnp.int32(_FLIP))


def _inv_key(key):
    bits = jnp.where(key >= 0, key, key ^ jnp.int32(_FLIP))
    return lax.bitcast_convert_type(bits, jnp.float32)


def _digit(key, shift):
    ub = key ^ jnp.int32(_MININT)
    return lax.shift_right_logical(ub, shift) & jnp.int32(0xFF)


def _body(scores_hbm, feats_hbm, out_hbm,
          scores_v, ck, ci, hist, summ, wl,
          gidx, gidx16, valsf, rows_t, sem):
    lanes = jnp.arange(_L, dtype=jnp.int32)
    zeros16 = jnp.zeros((_L,), jnp.int32)
    ones16 = jnp.ones((_L,), jnp.int32)
    b = lax.axis_index("c") * _NS + lax.axis_index("s")

    pltpu.sync_copy(
        scores_hbm.at[lax.div(b, jnp.int32(8)), :, lax.rem(b, jnp.int32(8)), :],
        scores_v)

    def zero_hist():
        def zbody(j, c):
            hist[pl.ds(j * _L, _L)] = zeros16
            return c
        lax.fori_loop(0, _NBINS, zbody, 0, unroll=8)

    def find_dstar(k_rem):
        def sbody(jj, carry):
            cum, dstar, above, found = carry
            j = _NBINS - 1 - jj
            tot = jnp.sum(hist[pl.ds(j * _L, _L)])
            new_cum = cum + tot
            crossed = jnp.logical_and(jnp.logical_not(found), new_cum >= k_rem)
            dstar = jnp.where(crossed, j, dstar)
            above = jnp.where(crossed, cum, above)
            found = jnp.logical_or(found, crossed)
            return (new_cum, dstar, above, found)
        _, dstar, above, _ = lax.fori_loop(
            0, _NBINS, sbody,
            (jnp.int32(0), jnp.int32(0), jnp.int32(0), False), unroll=4)
        return dstar, above

    zero_hist()

    @plsc.parallel_loop(0, 256, unroll=2)
    def _h0(r):
        for col in range(8):
            key = _mono_key(scores_v[r, pl.ds(col * _L, _L)])
            d = _digit(key, 24)
            plsc.addupdate_scatter(hist, [d * _L + lanes], ones16)
            cm = plsc.cummax(key)
            plsc.store_scatter(summ, [jnp.full((_L,), r * 8 + col, jnp.int32)],
                               cm, mask=lanes == _L - 1)

    d0star, above0 = find_dstar(jnp.int32(_K))
    k_rem = jnp.int32(_K) - above0

    @plsc.parallel_loop(0, _NV // _L, unroll=2, carry=zeros16)
    def nwl_v(w, nwl):
        mx = summ[pl.ds(w * _L, _L)]
        m = _digit(mx, 24) >= d0star
        kp = jnp.where(m, 1, 0)
        pos = nwl + plsc.cumsum(kp) - kp
        plsc.store_scatter(wl, [pos], w * _L + lanes, mask=m)
        return nwl + plsc.all_reduce_population_count(m)
    nwl = jnp.max(nwl_v)

    @plsc.parallel_loop(0, nwl, unroll=2, carry=zeros16)
    def cnt(w, cnt_c):
        vid = plsc.load_gather(wl, [jnp.full((_L,), w, jnp.int32)])
        row = lax.shift_right_logical(vid, 3)
        colb = (vid & jnp.int32(7)) * _L
        key = _mono_key(plsc.load_gather(scores_v, [row, colb + lanes]))
        m = _digit(key, 24) >= d0star
        addr = cnt_c * _L + lanes
        plsc.store_scatter(ck, [addr], key, mask=m)
        plsc.store_scatter(ci, [addr], vid * _L + lanes, mask=m)
        return cnt_c + jnp.where(m, 1, 0)
    nrows = jnp.max(cnt)

    zero_hist()

    @plsc.parallel_loop(0, nrows, unroll=2)
    def _h1(q):
        key = ck[pl.ds(q * _L, _L)]
        match = (q < cnt) & (_digit(key, 24) == d0star)
        plsc.addupdate_scatter(hist, [_digit(key, 16) * _L + lanes],
                               ones16, mask=match)

    d1star, above1 = find_dstar(k_rem)

    @plsc.parallel_loop(0, nrows, unroll=2, carry=zeros16)
    def m1_v(q, nglob):
        key = ck[pl.ds(q * _L, _L)]
        iv = ci[pl.ds(q * _L, _L)]
        d0 = _digit(key, 24)
        keep = (q < cnt) & ((d0 > d0star) |
                            ((d0 == d0star) & (_digit(key, 16) >= d1star)))
        kp = jnp.where(keep, 1, 0)
        pos = nglob + plsc.cumsum(kp) - kp
        plsc.store_scatter(ck, [pos], key, mask=keep)
        plsc.store_scatter(ci, [pos], iv, mask=keep)
        return nglob + plsc.all_reduce_population_count(keep)
    m1 = jnp.max(m1_v)
    nrows1 = lax.div(m1 + jnp.int32(_L - 1), jnp.int32(_L))

    _NR = 12

    def _rank_fast(_):
        krows = [ck[pl.ds(p * _L, _L)] for p in range(_NR)]
        irows = [ci[pl.ds(p * _L, _L)] for p in range(_NR)]

        @plsc.parallel_loop(0, m1, unroll=2,
                            carry=tuple(zeros16 for _ in range(_NR)))
        def ranks(j, rk):
            jv = jnp.full((_L,), j, jnp.int32)
            kj = plsc.load_gather(ck, [jv])
            ij = plsc.load_gather(ci, [jv])
            return tuple(
                rk[p] + jnp.where(
                    (kj > krows[p]) | ((kj == krows[p]) & (ij < irows[p])),
                    1, 0)
                for p in range(_NR))
        for p in range(_NR):
            mw = ((p * _L + lanes) < m1_v) & (ranks[p] < _K)
            plsc.store_scatter(gidx, [ranks[p]], irows[p], mask=mw)
            plsc.store_scatter(valsf, [ranks[p]], _inv_key(krows[p]), mask=mw)
        return 0

    def _rank_slow(_):
        @plsc.parallel_loop(0, m1, unroll=2)
        def _rank_one(s):
            sv = jnp.full((_L,), s, jnp.int32)
            kiv = plsc.load_gather(ck, [sv])
            iiv = plsc.load_gather(ci, [sv])

            def inner(q, rank):
                kq = ck[pl.ds(q * _L, _L)]
                iq = ci[pl.ds(q * _L, _L)]
                vq = (q * _L + lanes) < m1_v
                m = vq & ((kq > kiv) | ((kq == kiv) & (iq < iiv)))
                return rank + plsc.all_reduce_population_count(m)
            rank = lax.fori_loop(0, nrows1, inner, zeros16)

            mw = (rank < _K) & (lanes == 0)
            plsc.store_scatter(gidx, [rank], iiv, mask=mw)
            plsc.store_scatter(valsf, [rank], _inv_key(kiv), mask=mw)
        return 0

    lax.cond(m1 <= _NR * _L, _rank_fast, _rank_slow, 0)

    for blk in range(_K // _L):
        v = gidx[pl.ds(blk * _L, _L)]
        hi = lax.shift_left(lax.shift_right_logical(v, 7), 10) + \
            (v & jnp.int32(127))
        for j in range(_D):
            base = b * jnp.int32(1 << 19) + jnp.int32(
                (j // 8) * (1 << 18) + (j % 8) * 128)
            gidx16[pl.ds(j * _K + blk * _L, _L)] = hi + base
    descs = []
    for j in range(_D):
        descs.append(pltpu.async_copy(
            feats_hbm.at[gidx16.at[pl.ds(j * _K, _K)]],
            rows_t.at[pl.ds(j * _K, _K)], sem))
    for d in descs:
        d.wait()

    for j in range(_D):
        for blk in range(_K // _L):
            sl = pl.ds(j * _K + blk * _L, _L)
            rows_t[sl] = rows_t[sl] * valsf[pl.ds(blk * _L, _L)]

    pltpu.sync_copy(rows_t, out_hbm.at[pl.ds(b * _D * _K, _D * _K)])


@jax.jit
def _run(scores, feats2d):
    mesh = plsc.VectorSubcoreMesh(core_axis_name="c", subcore_axis_name="s",
                                  num_cores=_NC, num_subcores=_NS)
    fn = pl.kernel(
        _body,
        out_type=jax.ShapeDtypeStruct((_B * _D * _K,), jnp.float32),
        mesh=mesh,
        compiler_params=pltpu.CompilerParams(needs_layout_passes=False,
                                             use_tc_tiling_on_sc=False),
        scratch_types=[
            pltpu.VMEM((256, 128), jnp.float32),
            pltpu.VMEM((_N,), jnp.int32),
            pltpu.VMEM((_N,), jnp.int32),
            pltpu.VMEM((_NBINS * _L,), jnp.int32),
            pltpu.VMEM((_NV,), jnp.int32),
            pltpu.VMEM((_NV,), jnp.int32),
            pltpu.VMEM((_K,), jnp.int32),
            pltpu.VMEM((_D * _K,), jnp.int32),
            pltpu.VMEM((_K,), jnp.float32),
            pltpu.VMEM((_D * _K,), jnp.float32),
            pltpu.SemaphoreType.DMA,
        ],
    )
    return fn(scores, feats2d)


def kernel(scores, score_feats, k):
    del k
    scores_t = scores.reshape(4, 8, 256, 128).transpose(0, 2, 1, 3)
    feats_l = (score_feats.transpose(0, 2, 1)
               .reshape(_B, 2, 8, 256, 128)
               .transpose(0, 1, 3, 2, 4)
               .reshape(-1))
    flat = _run(scores_t, feats_l)
    return flat.reshape(_B, _D, _K).transpose(0, 2, 1)
